# Initial kernel scaffold; baseline (speedup 1.0000x reference)
import jax
import jax.numpy as jnp
from jax.experimental import pallas as pl


def kernel(x, edge_index, W1, a_src1, a_dst1, b1, W2, a_src2, a_dst2, b2):
    raise NotImplementedError("write your pallas kernel here")


# speedup vs baseline: 19.9346x; 19.9346x over previous
import dataclasses
import functools

import jax
import jax.numpy as jnp
from jax import lax
from jax.experimental import pallas as pl
from jax.experimental.pallas import tpu as pltpu
from jax.experimental.pallas import tpu_sc as plsc

_N = 10000
_E = 320000
_D = 128
_H1 = 12
_O1 = 16
_OUT = 128

_W1COL = _H1 * _O1
_HH = _H1 // 2
_HW = _HH * _O1

_NC = 2
_NS = 16
_B = 80
_RC = 80
_NCH = _N // _RC
_CPT = (_NCH + _NS - 1) // _NS
_ND = 1280

_RB = 1000


def _leaky_exp(e):
    return jnp.exp(jnp.maximum(e, 0.2 * e))


def _dot(a, b):
    return jnp.dot(a, b, precision=lax.Precision.HIGHEST,
                   preferred_element_type=jnp.float32)


def _sc_compiler_params():
    cp = pltpu.CompilerParams()
    if "needs_layout_passes" in pltpu.CompilerParams.__dataclass_fields__:
        cp = dataclasses.replace(cp, needs_layout_passes=False)
    return cp


def _tc_prep1(x, W1, A1s, A1d, A1s_lo, A1s_hi, A1d_lo, A1d_hi):
    def body(x_ref, w_ref, asf_ref, adf_ref, aslo_ref, ashi_ref, adlo_ref,
             adhi_ref, h_ref, t1lo_ref, t1hi_ref, ad6lo_ref, ad6hi_ref,
             asp_ref, adp_ref):
        h = _dot(x_ref[...], w_ref[...])
        h_ref[...] = h
        asp_ref[...] = _dot(h, asf_ref[...])
        adp_ref[...] = _dot(h, adf_ref[...])
        z16 = jnp.zeros((h.shape[0], 16), jnp.float32)
        aslo = _dot(h, aslo_ref[...])
        ashi = _dot(h, ashi_ref[...])
        t1lo_ref[...] = jnp.concatenate([h[:, :_HW], aslo, z16], axis=1)
        t1hi_ref[...] = jnp.concatenate([h[:, _HW:_W1COL], ashi, z16], axis=1)
        ad6lo_ref[...] = _dot(h, adlo_ref[...])
        ad6hi_ref[...] = _dot(h, adhi_ref[...])

    full = lambda shape: pl.BlockSpec(shape, lambda i: (0, 0))
    row = lambda cols: pl.BlockSpec((_RB, cols), lambda i: (i, 0))
    return pl.pallas_call(
        body,
        grid=(_N // _RB,),
        in_specs=[row(_D), full((_D, _W1COL)), full((_W1COL, 16)),
                  full((_W1COL, 16)), full((_W1COL, 16)), full((_W1COL, 16)),
                  full((_W1COL, 8)), full((_W1COL, 8))],
        out_specs=[row(_W1COL), row(128), row(128), row(8), row(8),
                   row(16), row(16)],
        out_shape=[
            jax.ShapeDtypeStruct((_N, _W1COL), jnp.float32),
            jax.ShapeDtypeStruct((_N, 128), jnp.float32),
            jax.ShapeDtypeStruct((_N, 128), jnp.float32),
            jax.ShapeDtypeStruct((_N, 8), jnp.float32),
            jax.ShapeDtypeStruct((_N, 8), jnp.float32),
            jax.ShapeDtypeStruct((_N, 16), jnp.float32),
            jax.ShapeDtypeStruct((_N, 16), jnp.float32),
        ],
    )(x, W1, A1s, A1d, A1s_lo, A1s_hi, A1d_lo, A1d_hi)


def _tc_finish1(ulo, uhi, h1, asp, adp, R, b1r, W2, a2cols):
    def body(ulo_ref, uhi_ref, h_ref, as_ref, ad_ref, r_ref, b_ref, w2_ref,
             a2_ref, h2_ref, sa_ref):
        e = as_ref[...] + ad_ref[...]
        exl = _leaky_exp(e)
        rm = r_ref[...]
        ulob = ulo_ref[...]
        uhib = uhi_ref[...]
        dent = jnp.concatenate(
            [ulob[:, _HW:_HW + _HH], uhib[:, _HW:_HW + _HH],
             jnp.zeros((ulob.shape[0], 4), jnp.float32)], axis=1) + exl
        numer = (jnp.concatenate([ulob[:, :_HW], uhib[:, :_HW]], axis=1)
                 + h_ref[...] * _dot(exl, rm))
        out1 = numer / (_dot(dent, rm)
                        + 1e-16) + b_ref[...]
        x2 = jnp.maximum(out1, 0.0)
        h2 = _dot(x2, w2_ref[...])
        h2_ref[...] = h2
        sa_ref[...] = _dot(h2, a2_ref[...])

    full = lambda shape: pl.BlockSpec(shape, lambda i: (0, 0))
    row = lambda cols: pl.BlockSpec((_RB, cols), lambda i: (i, 0))
    return pl.pallas_call(
        body,
        grid=(_N // _RB,),
        in_specs=[row(128), row(128), row(_W1COL), row(16), row(16),
                  full((16, _W1COL)),
                  pl.BlockSpec((1, _W1COL), lambda i: (0, 0)),
                  full((_W1COL, _OUT)), full((_OUT, 8))],
        out_specs=[row(_OUT), row(8)],
        out_shape=[
            jax.ShapeDtypeStruct((_N, _OUT), jnp.float32),
            jax.ShapeDtypeStruct((_N, 8), jnp.float32),
        ],
    )(ulo, uhi, h1, asp, adp, R, b1r, W2, a2cols)


def _tc_finish2(u20, u21, ud0, ud1, h2, sa2, b2r):
    def body(u20_ref, u21_ref, ud0_ref, ud1_ref, h2_ref, sa_ref, b_ref,
             out_ref):
        sa = sa_ref[...]
        e = sa[:, 0:1] + sa[:, 1:2]
        exl = _leaky_exp(e)
        denom = ud0_ref[:, 0:1] + ud1_ref[:, 0:1] + exl + 1e-16
        numer = u20_ref[...] + u21_ref[...] + h2_ref[...] * exl
        out_ref[...] = numer / denom + b_ref[...]

    row = lambda cols: pl.BlockSpec((_RB, cols), lambda i: (i, 0))
    return pl.pallas_call(
        body,
        grid=(_N // _RB,),
        in_specs=[row(_OUT), row(_OUT), row(16), row(16), row(_OUT),
                  row(8), pl.BlockSpec((1, _OUT), lambda i: (0, 0))],
        out_specs=row(_OUT),
        out_shape=jax.ShapeDtypeStruct((_N, _OUT), jnp.float32),
    )(u20, u21, ud0, ud1, h2, sa2, b2r)


def _sc_edge_pass1(src, dst, t1, ad_all):
    mesh = plsc.VectorSubcoreMesh(core_axis_name="c", subcore_axis_name="s")
    ept = _E // _NS
    nb = ept // _B

    @functools.partial(
        pl.kernel, mesh=mesh,
        compiler_params=_sc_compiler_params(),
        out_type=jax.ShapeDtypeStruct((_NC * _N, 128), jnp.float32),
        scratch_types=[
            pltpu.VMEM_SHARED((_N, 128), jnp.float32),
            pltpu.VMEM((3 * _N,), jnp.int32),
            pltpu.VMEM((_B,), jnp.int32),
            pltpu.VMEM((_B,), jnp.int32),
            pltpu.VMEM((_B,), jnp.int32),
            pltpu.VMEM((_B, 128), jnp.float32),
            pltpu.VMEM((_B, 16), jnp.float32),
        ])
    def k(src_hbm, dst_hbm, t1_hbm, ad_hbm, u_hbm,
          acc, ad32, srcb, srcb2, dstb, msgb, exband):
        sc = lax.axis_index("c")
        tid = lax.axis_index("s")

        pltpu.sync_copy(ad_hbm.at[pl.ds(sc * (3 * _N), 3 * _N)], ad32)

        @pl.loop(0, _RC)
        def _(r):
            for c in range(8):
                msgb[r, pl.ds(c * 16, 16)] = jnp.zeros((16,), jnp.float32)

        @pl.loop(0, _CPT)
        def _(kk):
            ch = kk * _NS + tid

            @pl.when(ch < _NCH)
            def _():
                pltpu.sync_copy(msgb, acc.at[pl.ds(ch * _RC, _RC)])

        plsc.subcore_barrier()

        lanes = lax.iota(jnp.int32, 16)
        offs3 = jnp.minimum(lax.shift_right_logical(lanes, 1), 2)
        even = (lanes & 1) == 0
        himask = jnp.full((16,), -65536, jnp.int32)
        base = tid * ept

        rowoff = jnp.full((16,), sc * _N, jnp.int32)

        @pl.loop(0, nb)
        def _(blk):
            off = base + blk * _B
            pltpu.sync_copy(src_hbm.at[pl.ds(off, _B)], srcb)
            pltpu.sync_copy(dst_hbm.at[pl.ds(off, _B)], dstb)

            for c in range(_B // 16):
                srcb2[pl.ds(c * 16, 16)] = srcb[pl.ds(c * 16, 16)] + rowoff

            pltpu.sync_copy(t1_hbm.at[srcb2], msgb)

            @pl.loop(0, _B)
            def _(i):
                asv = msgb[i, pl.ds(_HW, 16)]
                dvb = plsc.load_gather(dstb, [jnp.full((16,), i, jnp.int32)])
                g = plsc.load_gather(ad32, [dvb * 3 + offs3])
                bits = jnp.where(even, g & himask,
                                 lax.shift_left(g, jnp.full((16,), 16,
                                                            jnp.int32)))
                adv = lax.bitcast_convert_type(bits, jnp.float32)
                e = asv + adv
                ex = _leaky_exp(e)
                exband[i, :] = ex
                msgb[i, pl.ds(_HW, 16)] = ex

            @pl.loop(0, _B)
            def _(i):
                iv = jnp.full((16,), i, jnp.int32)
                for hd in range(_HH):
                    bc = plsc.load_gather(
                        exband, [iv, jnp.full((16,), hd, jnp.int32)])
                    msgb[i, pl.ds(hd * 16, 16)] = (
                        msgb[i, pl.ds(hd * 16, 16)] * bc)

            pltpu.sync_copy(msgb, acc.at[dstb], add=True)

        plsc.subcore_barrier()

        @pl.loop(0, _CPT)
        def _(kk):
            ch = kk * _NS + tid

            @pl.when(ch < _NCH)
            def _():
                r0 = ch * _RC
                pltpu.sync_copy(acc.at[pl.ds(r0, _RC)], msgb)
                pltpu.sync_copy(msgb, u_hbm.at[pl.ds(sc * _N + r0, _RC)])

    return k(src, dst, t1, ad_all)


def _sc_edge_pass2(src, dst, sa32, h2):
    mesh = plsc.VectorSubcoreMesh(core_axis_name="c", subcore_axis_name="s")
    ept = _E // (_NC * _NS)
    nb = ept // _B

    @functools.partial(
        pl.kernel, mesh=mesh,
        compiler_params=_sc_compiler_params(),
        out_type=[jax.ShapeDtypeStruct((_NC * _N, 128), jnp.float32),
                  jax.ShapeDtypeStruct((_NC * _ND, 128), jnp.float32)],
        scratch_types=[
            pltpu.VMEM_SHARED((_N, 128), jnp.float32),
            pltpu.VMEM_SHARED((_ND, 128), jnp.float32),
            pltpu.VMEM((_N,), jnp.int32),
            pltpu.VMEM((_B,), jnp.int32),
            pltpu.VMEM((_B,), jnp.int32),
            pltpu.VMEM((_B,), jnp.int32),
            pltpu.VMEM((_B,), jnp.float32),
            pltpu.VMEM((_B, 128), jnp.float32),
            pltpu.VMEM((_B, 128), jnp.float32),
        ])
    def k(src_hbm, dst_hbm, sa_hbm, h_hbm, u_hbm, ud_hbm,
          acc, accd, sa32v, srcb, dstb, dstq, exb, msgb, msgd):
        sc = lax.axis_index("c")
        tid = lax.axis_index("s")

        pltpu.sync_copy(sa_hbm, sa32v)

        @pl.loop(0, _RC)
        def _(r):
            for c in range(8):
                msgb[r, pl.ds(c * 16, 16)] = jnp.zeros((16,), jnp.float32)

        @pl.loop(0, _CPT)
        def _(kk):
            ch = kk * _NS + tid

            @pl.when(ch < _NCH)
            def _():
                pltpu.sync_copy(msgb, acc.at[pl.ds(ch * _RC, _RC)])

        pltpu.sync_copy(msgb, accd.at[pl.ds(tid * _RC, _RC)])

        plsc.subcore_barrier()

        himask = jnp.full((16,), -65536, jnp.int32)
        sixteen = jnp.full((16,), 16, jnp.int32)
        base = (sc * _NS + tid) * ept

        @pl.loop(0, nb)
        def _(blk):
            off = base + blk * _B
            pltpu.sync_copy(src_hbm.at[pl.ds(off, _B)], srcb)
            pltpu.sync_copy(dst_hbm.at[pl.ds(off, _B)], dstb)
            pltpu.sync_copy(h_hbm.at[srcb], msgb)

            for c in range(_B // 16):
                sv = srcb[pl.ds(c * 16, 16)]
                dv = dstb[pl.ds(c * 16, 16)]
                gs = plsc.load_gather(sa32v, [sv])
                gd = plsc.load_gather(sa32v, [dv])
                e = (lax.bitcast_convert_type(gs & himask, jnp.float32)
                     + lax.bitcast_convert_type(lax.shift_left(gd, sixteen),
                                                jnp.float32))
                exb[pl.ds(c * 16, 16)] = _leaky_exp(e)
                dstq[pl.ds(c * 16, 16)] = lax.shift_right_logical(dv, 3)

            @pl.loop(0, _B)
            def _(i):
                iv = jnp.full((16,), i, jnp.int32)
                bc = plsc.load_gather(exb, [iv])
                for c in range(_OUT // 16):
                    msgb[i, pl.ds(c * 16, 16)] = (
                        msgb[i, pl.ds(c * 16, 16)] * bc)
                dvb = plsc.load_gather(dstb, [iv])
                slot = (lax.reduce_max(dvb, axes=(0,)) & 7) * 16
                for c in range(8):
                    msgd[i, pl.ds(c * 16, 16)] = jnp.zeros((16,), jnp.float32)
                msgd[i, pl.ds(slot, 16)] = bc

            pltpu.sync_copy(msgb, acc.at[dstb], add=True)
            pltpu.sync_copy(msgd, accd.at[dstq], add=True)

        plsc.subcore_barrier()

        @pl.loop(0, _CPT)
        def _(kk):
            ch = kk * _NS + tid

            @pl.when(ch < _NCH)
            def _():
                r0 = ch * _RC
                pltpu.sync_copy(acc.at[pl.ds(r0, _RC)], msgb)
                pltpu.sync_copy(msgb, u_hbm.at[pl.ds(sc * _N + r0, _RC)])

        r0 = tid * _RC
        pltpu.sync_copy(accd.at[pl.ds(r0, _RC)], msgb)
        pltpu.sync_copy(msgb, ud_hbm.at[pl.ds(sc * _ND + r0, _RC)])

    return k(src, dst, sa32, h2)


def kernel(x, edge_index, W1, a_src1, a_dst1, b1, W2, a_src2, a_dst2, b2):
    return _run(x, edge_index, W1, a_src1, a_dst1, b1, W2, a_src2, a_dst2,
                b2)[-1]


def _run(x, edge_index, W1, a_src1, a_dst1, b1, W2, a_src2, a_dst2, b2):
    src = edge_index[0]
    dst = edge_index[1]

    rows = jnp.arange(_W1COL, dtype=jnp.int32)
    head_of_row = rows // _O1
    lane16 = jnp.arange(16, dtype=jnp.int32)

    def attn_mat(a, lo, hi, width):
        m = jnp.zeros((_W1COL, width), jnp.float32)
        sel = (head_of_row >= lo) & (head_of_row < hi)
        col = jnp.where(sel, head_of_row - lo, width - 1)
        val = jnp.where(sel, a.reshape(-1), 0.0)
        return m.at[rows, col].add(val)

    A1s = attn_mat(a_src1, 0, _H1, 16)
    A1d = attn_mat(a_dst1, 0, _H1, 16)
    A1s_lo = attn_mat(a_src1, 0, _HH, 16)
    A1s_hi = attn_mat(a_src1, _HH, _H1, 16)
    A1d_lo = attn_mat(a_dst1, 0, _HH, 8)
    A1d_hi = attn_mat(a_dst1, _HH, _H1, 8)
    R = (lane16[:, None] == head_of_row[None, :]).astype(jnp.float32)
    a2cols = jnp.zeros((_OUT, 8), jnp.float32)
    a2cols = a2cols.at[:, 0].set(a_src2[0]).at[:, 1].set(a_dst2[0])
    b1r = b1.reshape(1, _W1COL)
    b2r = b2.reshape(1, _OUT)

    def pack_pairs(m6):
        b = lax.bitcast_convert_type(m6.astype(jnp.bfloat16),
                                     jnp.uint16).astype(jnp.uint32)
        w = (b[:, 0::2] << 16) | b[:, 1::2]
        return lax.bitcast_convert_type(w, jnp.int32).reshape(-1)

    h1, t1lo, t1hi, ad6lo, ad6hi, asp, adp = _tc_prep1(
        x, W1, A1s, A1d, A1s_lo, A1s_hi, A1d_lo, A1d_hi)
    t1 = jnp.concatenate([t1lo, t1hi], axis=0)
    ad_all = jnp.concatenate([pack_pairs(ad6lo[:, :_HH]),
                              pack_pairs(ad6hi[:, :_HH])])
    u1f = _sc_edge_pass1(src, dst, t1, ad_all)
    u1 = (u1f[:_N], u1f[_N:])
    h2, sa2 = _tc_finish1(u1[0], u1[1], h1, asp, adp, R, b1r, W2, a2cols)
    sab = lax.bitcast_convert_type(sa2[:, :2].astype(jnp.bfloat16),
                                   jnp.uint16).astype(jnp.uint32)
    sa32 = lax.bitcast_convert_type((sab[:, 0] << 16) | sab[:, 1], jnp.int32)
    u2f, ud2f = _sc_edge_pass2(src, dst, sa32, h2)
    u2 = (u2f[:_N], u2f[_N:])
    ud2 = (ud2f[:_ND], ud2f[_ND:])
    ud0 = ud2[0].reshape(_ND * 8, 16)[:_N]
    ud1 = ud2[1].reshape(_ND * 8, 16)[:_N]
    out = _tc_finish2(u2[0], u2[1], ud0, ud1, h2, sa2, b2r)
    return (h1, t1lo, t1hi, ad6lo, ad6hi, asp, adp, u1, h2, sa2, sa32,
            u2, ud2, out)


# speedup vs baseline: 22.1820x; 1.1127x over previous
import dataclasses
import functools

import jax
import jax.numpy as jnp
from jax import lax
from jax.experimental import pallas as pl
from jax.experimental.pallas import tpu as pltpu
from jax.experimental.pallas import tpu_sc as plsc

_N = 10000
_E = 320000
_D = 128
_H1 = 12
_O1 = 16
_OUT = 128

_W1COL = _H1 * _O1
_HH = _H1 // 2
_HW = _HH * _O1

_NC = 2
_NS = 16
_B = 80
_RC = 80
_NCH = _N // _RC
_CPT = (_NCH + _NS - 1) // _NS
_ND = 1280

_RB = 1000


def _leaky_exp(e):
    return jnp.exp(jnp.maximum(e, 0.2 * e))


def _dot(a, b):
    return jnp.dot(a, b, precision=lax.Precision.HIGHEST,
                   preferred_element_type=jnp.float32)


def _sc_compiler_params():
    cp = pltpu.CompilerParams()
    if "needs_layout_passes" in pltpu.CompilerParams.__dataclass_fields__:
        cp = dataclasses.replace(cp, needs_layout_passes=False)
    return cp


def _tc_prep1(x, W1, A1s, A1d, A1s_lo, A1s_hi, A1d_lo, A1d_hi):
    def body(x_ref, w_ref, asf_ref, adf_ref, aslo_ref, ashi_ref, adlo_ref,
             adhi_ref, h_ref, t1lo_ref, t1hi_ref, ad6lo_ref, ad6hi_ref,
             asp_ref, adp_ref):
        h = _dot(x_ref[...], w_ref[...])
        h_ref[...] = h
        asp_ref[...] = _dot(h, asf_ref[...])
        adp_ref[...] = _dot(h, adf_ref[...])
        z16 = jnp.zeros((h.shape[0], 16), jnp.float32)
        aslo = _dot(h, aslo_ref[...])
        ashi = _dot(h, ashi_ref[...])
        t1lo_ref[...] = jnp.concatenate([h[:, :_HW], aslo, z16], axis=1)
        t1hi_ref[...] = jnp.concatenate([h[:, _HW:_W1COL], ashi, z16], axis=1)
        ad6lo_ref[...] = _dot(h, adlo_ref[...])
        ad6hi_ref[...] = _dot(h, adhi_ref[...])

    full = lambda shape: pl.BlockSpec(shape, lambda i: (0, 0))
    row = lambda cols: pl.BlockSpec((_RB, cols), lambda i: (i, 0))
    return pl.pallas_call(
        body,
        grid=(_N // _RB,),
        in_specs=[row(_D), full((_D, _W1COL)), full((_W1COL, 16)),
                  full((_W1COL, 16)), full((_W1COL, 16)), full((_W1COL, 16)),
                  full((_W1COL, 8)), full((_W1COL, 8))],
        out_specs=[row(_W1COL), row(128), row(128), row(8), row(8),
                   row(16), row(16)],
        out_shape=[
            jax.ShapeDtypeStruct((_N, _W1COL), jnp.float32),
            jax.ShapeDtypeStruct((_N, 128), jnp.float32),
            jax.ShapeDtypeStruct((_N, 128), jnp.float32),
            jax.ShapeDtypeStruct((_N, 8), jnp.float32),
            jax.ShapeDtypeStruct((_N, 8), jnp.float32),
            jax.ShapeDtypeStruct((_N, 16), jnp.float32),
            jax.ShapeDtypeStruct((_N, 16), jnp.float32),
        ],
    )(x, W1, A1s, A1d, A1s_lo, A1s_hi, A1d_lo, A1d_hi)


def _tc_finish1(ulo, uhi, h1, asp, adp, R, b1r, W2, a2cols):
    def body(ulo_ref, uhi_ref, h_ref, as_ref, ad_ref, r_ref, b_ref, w2_ref,
             a2_ref, h2_ref, sa_ref):
        e = as_ref[...] + ad_ref[...]
        exl = _leaky_exp(e)
        rm = r_ref[...]
        ulob = ulo_ref[...]
        uhib = uhi_ref[...]
        dent = jnp.concatenate(
            [ulob[:, _HW:_HW + _HH], uhib[:, _HW:_HW + _HH],
             jnp.zeros((ulob.shape[0], 4), jnp.float32)], axis=1) + exl
        numer = (jnp.concatenate([ulob[:, :_HW], uhib[:, :_HW]], axis=1)
                 + h_ref[...] * _dot(exl, rm))
        out1 = numer / (_dot(dent, rm)
                        + 1e-16) + b_ref[...]
        x2 = jnp.maximum(out1, 0.0)
        h2 = _dot(x2, w2_ref[...])
        h2_ref[...] = h2
        sa_ref[...] = _dot(h2, a2_ref[...])

    full = lambda shape: pl.BlockSpec(shape, lambda i: (0, 0))
    row = lambda cols: pl.BlockSpec((_RB, cols), lambda i: (i, 0))
    return pl.pallas_call(
        body,
        grid=(_N // _RB,),
        in_specs=[row(128), row(128), row(_W1COL), row(16), row(16),
                  full((16, _W1COL)),
                  pl.BlockSpec((1, _W1COL), lambda i: (0, 0)),
                  full((_W1COL, _OUT)), full((_OUT, 8))],
        out_specs=[row(_OUT), row(8)],
        out_shape=[
            jax.ShapeDtypeStruct((_N, _OUT), jnp.float32),
            jax.ShapeDtypeStruct((_N, 8), jnp.float32),
        ],
    )(ulo, uhi, h1, asp, adp, R, b1r, W2, a2cols)


def _tc_finish2(u20, u21, ud0, ud1, h2, sa2, b2r):
    def body(u20_ref, u21_ref, ud0_ref, ud1_ref, h2_ref, sa_ref, b_ref,
             out_ref):
        sa = sa_ref[...]
        e = sa[:, 0:1] + sa[:, 1:2]
        exl = _leaky_exp(e)
        denom = ud0_ref[:, 0:1] + ud1_ref[:, 0:1] + exl + 1e-16
        numer = u20_ref[...] + u21_ref[...] + h2_ref[...] * exl
        out_ref[...] = numer / denom + b_ref[...]

    row = lambda cols: pl.BlockSpec((_RB, cols), lambda i: (i, 0))
    return pl.pallas_call(
        body,
        grid=(_N // _RB,),
        in_specs=[row(_OUT), row(_OUT), row(16), row(16), row(_OUT),
                  row(8), pl.BlockSpec((1, _OUT), lambda i: (0, 0))],
        out_specs=row(_OUT),
        out_shape=jax.ShapeDtypeStruct((_N, _OUT), jnp.float32),
    )(u20, u21, ud0, ud1, h2, sa2, b2r)


def _sc_edge_pass1(src, dst, t1, ad_all):
    mesh = plsc.VectorSubcoreMesh(core_axis_name="c", subcore_axis_name="s")
    ept = _E // _NS
    nb = ept // _B

    @functools.partial(
        pl.kernel, mesh=mesh,
        compiler_params=_sc_compiler_params(),
        out_type=jax.ShapeDtypeStruct((_NC * _N, 128), jnp.float32),
        scratch_types=[
            pltpu.VMEM_SHARED((_N, 128), jnp.float32),
            pltpu.VMEM((3 * _N,), jnp.int32),
            pltpu.VMEM((_B,), jnp.int32),
            pltpu.VMEM((_B,), jnp.int32),
            pltpu.VMEM((_B,), jnp.int32),
            pltpu.VMEM((_B, 128), jnp.float32),
            pltpu.VMEM((_B * 16,), jnp.float32),
        ])
    def k(src_hbm, dst_hbm, t1_hbm, ad_hbm, u_hbm,
          acc, ad32, srcb, srcb2, dstb, msgb, exflat):
        sc = lax.axis_index("c")
        tid = lax.axis_index("s")

        pltpu.sync_copy(ad_hbm.at[pl.ds(sc * (3 * _N), 3 * _N)], ad32)

        @pl.loop(0, _RC)
        def _(r):
            for c in range(8):
                msgb[r, pl.ds(c * 16, 16)] = jnp.zeros((16,), jnp.float32)

        @pl.loop(0, _CPT)
        def _(kk):
            ch = kk * _NS + tid

            @pl.when(ch < _NCH)
            def _():
                pltpu.sync_copy(msgb, acc.at[pl.ds(ch * _RC, _RC)])

        plsc.subcore_barrier()

        lanes = lax.iota(jnp.int32, 16)
        lanes16 = lanes * 16
        hm = jnp.minimum(lanes, _HH - 1)
        himask = jnp.full((16,), -65536, jnp.int32)
        sixteen = jnp.full((16,), 16, jnp.int32)
        base = tid * ept

        rowoff = jnp.full((16,), sc * _N, jnp.int32)

        @pl.loop(0, nb)
        def _(blk):
            off = base + blk * _B
            pltpu.sync_copy(src_hbm.at[pl.ds(off, _B)], srcb)
            pltpu.sync_copy(dst_hbm.at[pl.ds(off, _B)], dstb)

            for c in range(_B // 16):
                srcb2[pl.ds(c * 16, 16)] = srcb[pl.ds(c * 16, 16)] + rowoff

            pltpu.sync_copy(t1_hbm.at[srcb2], msgb)

            for c in range(_B // 16):
                ev = lanes + c * 16
                dv3 = dstb[pl.ds(c * 16, 16)] * 3
                for kw in range(3):
                    gd = plsc.load_gather(ad32, [dv3 + kw])
                    adhi = lax.bitcast_convert_type(gd & himask, jnp.float32)
                    adlo = lax.bitcast_convert_type(
                        lax.shift_left(gd, sixteen), jnp.float32)
                    ashi = plsc.load_gather(
                        msgb, [ev, jnp.full((16,), _HW + 2 * kw, jnp.int32)])
                    aslo = plsc.load_gather(
                        msgb,
                        [ev, jnp.full((16,), _HW + 2 * kw + 1, jnp.int32)])
                    plsc.store_scatter(exflat, [lanes16 + (c * 256 + 2 * kw)],
                                       _leaky_exp(ashi + adhi))
                    plsc.store_scatter(exflat,
                                       [lanes16 + (c * 256 + 2 * kw + 1)],
                                       _leaky_exp(aslo + adlo))

            @pl.loop(0, _B)
            def _(i):
                b16 = i * 16
                msgb[i, pl.ds(_HW, 16)] = plsc.load_gather(exflat, [hm + b16])
                for hd in range(_HH):
                    bc = plsc.load_gather(
                        exflat, [jnp.full((16,), b16 + hd, jnp.int32)])
                    msgb[i, pl.ds(hd * 16, 16)] = (
                        msgb[i, pl.ds(hd * 16, 16)] * bc)

            pltpu.sync_copy(msgb, acc.at[dstb], add=True)

        plsc.subcore_barrier()

        @pl.loop(0, _CPT)
        def _(kk):
            ch = kk * _NS + tid

            @pl.when(ch < _NCH)
            def _():
                r0 = ch * _RC
                pltpu.sync_copy(acc.at[pl.ds(r0, _RC)], msgb)
                pltpu.sync_copy(msgb, u_hbm.at[pl.ds(sc * _N + r0, _RC)])

    return k(src, dst, t1, ad_all)


def _sc_edge_pass2(src, dst, sa32, h2):
    mesh = plsc.VectorSubcoreMesh(core_axis_name="c", subcore_axis_name="s")
    ept = _E // (_NC * _NS)
    nb = ept // _B

    @functools.partial(
        pl.kernel, mesh=mesh,
        compiler_params=_sc_compiler_params(),
        out_type=[jax.ShapeDtypeStruct((_NC * _N, 128), jnp.float32),
                  jax.ShapeDtypeStruct((_NC * _ND, 128), jnp.float32)],
        scratch_types=[
            pltpu.VMEM_SHARED((_N, 128), jnp.float32),
            pltpu.VMEM_SHARED((_ND, 128), jnp.float32),
            pltpu.VMEM((_N,), jnp.int32),
            pltpu.VMEM((_B,), jnp.int32),
            pltpu.VMEM((_B,), jnp.int32),
            pltpu.VMEM((_B,), jnp.int32),
            pltpu.VMEM((_B,), jnp.float32),
            pltpu.VMEM((_B, 128), jnp.float32),
            pltpu.VMEM((_B, 128), jnp.float32),
        ])
    def k(src_hbm, dst_hbm, sa_hbm, h_hbm, u_hbm, ud_hbm,
          acc, accd, sa32v, srcb, dstb, dstq, exb, msgb, msgd):
        sc = lax.axis_index("c")
        tid = lax.axis_index("s")

        pltpu.sync_copy(sa_hbm, sa32v)

        @pl.loop(0, _RC)
        def _(r):
            for c in range(8):
                msgb[r, pl.ds(c * 16, 16)] = jnp.zeros((16,), jnp.float32)

        @pl.loop(0, _CPT)
        def _(kk):
            ch = kk * _NS + tid

            @pl.when(ch < _NCH)
            def _():
                pltpu.sync_copy(msgb, acc.at[pl.ds(ch * _RC, _RC)])

        pltpu.sync_copy(msgb, accd.at[pl.ds(tid * _RC, _RC)])

        plsc.subcore_barrier()

        himask = jnp.full((16,), -65536, jnp.int32)
        sixteen = jnp.full((16,), 16, jnp.int32)
        base = (sc * _NS + tid) * ept

        @pl.loop(0, nb)
        def _(blk):
            off = base + blk * _B
            pltpu.sync_copy(src_hbm.at[pl.ds(off, _B)], srcb)
            pltpu.sync_copy(dst_hbm.at[pl.ds(off, _B)], dstb)
            pltpu.sync_copy(h_hbm.at[srcb], msgb)

            for c in range(_B // 16):
                sv = srcb[pl.ds(c * 16, 16)]
                dv = dstb[pl.ds(c * 16, 16)]
                gs = plsc.load_gather(sa32v, [sv])
                gd = plsc.load_gather(sa32v, [dv])
                e = (lax.bitcast_convert_type(gs & himask, jnp.float32)
                     + lax.bitcast_convert_type(lax.shift_left(gd, sixteen),
                                                jnp.float32))
                exb[pl.ds(c * 16, 16)] = _leaky_exp(e)
                dstq[pl.ds(c * 16, 16)] = lax.shift_right_logical(dv, 3)

            @pl.loop(0, _B)
            def _(i):
                iv = jnp.full((16,), i, jnp.int32)
                bc = plsc.load_gather(exb, [iv])
                for c in range(_OUT // 16):
                    msgb[i, pl.ds(c * 16, 16)] = (
                        msgb[i, pl.ds(c * 16, 16)] * bc)
                dvb = plsc.load_gather(dstb, [iv])
                slot = (lax.reduce_max(dvb, axes=(0,)) & 7) * 16
                for c in range(8):
                    msgd[i, pl.ds(c * 16, 16)] = jnp.zeros((16,), jnp.float32)
                msgd[i, pl.ds(slot, 16)] = bc

            pltpu.sync_copy(msgb, acc.at[dstb], add=True)
            pltpu.sync_copy(msgd, accd.at[dstq], add=True)

        plsc.subcore_barrier()

        @pl.loop(0, _CPT)
        def _(kk):
            ch = kk * _NS + tid

            @pl.when(ch < _NCH)
            def _():
                r0 = ch * _RC
                pltpu.sync_copy(acc.at[pl.ds(r0, _RC)], msgb)
                pltpu.sync_copy(msgb, u_hbm.at[pl.ds(sc * _N + r0, _RC)])

        r0 = tid * _RC
        pltpu.sync_copy(accd.at[pl.ds(r0, _RC)], msgb)
        pltpu.sync_copy(msgb, ud_hbm.at[pl.ds(sc * _ND + r0, _RC)])

    return k(src, dst, sa32, h2)


def kernel(x, edge_index, W1, a_src1, a_dst1, b1, W2, a_src2, a_dst2, b2):
    return _run(x, edge_index, W1, a_src1, a_dst1, b1, W2, a_src2, a_dst2,
                b2)[-1]


def _run(x, edge_index, W1, a_src1, a_dst1, b1, W2, a_src2, a_dst2, b2):
    src = edge_index[0]
    dst = edge_index[1]

    rows = jnp.arange(_W1COL, dtype=jnp.int32)
    head_of_row = rows // _O1
    lane16 = jnp.arange(16, dtype=jnp.int32)

    def attn_mat(a, lo, hi, width):
        m = jnp.zeros((_W1COL, width), jnp.float32)
        sel = (head_of_row >= lo) & (head_of_row < hi)
        col = jnp.where(sel, head_of_row - lo, width - 1)
        val = jnp.where(sel, a.reshape(-1), 0.0)
        return m.at[rows, col].add(val)

    A1s = attn_mat(a_src1, 0, _H1, 16)
    A1d = attn_mat(a_dst1, 0, _H1, 16)
    A1s_lo = attn_mat(a_src1, 0, _HH, 16)
    A1s_hi = attn_mat(a_src1, _HH, _H1, 16)
    A1d_lo = attn_mat(a_dst1, 0, _HH, 8)
    A1d_hi = attn_mat(a_dst1, _HH, _H1, 8)
    R = (lane16[:, None] == head_of_row[None, :]).astype(jnp.float32)
    a2cols = jnp.zeros((_OUT, 8), jnp.float32)
    a2cols = a2cols.at[:, 0].set(a_src2[0]).at[:, 1].set(a_dst2[0])
    b1r = b1.reshape(1, _W1COL)
    b2r = b2.reshape(1, _OUT)

    def pack_pairs(m6):
        b = lax.bitcast_convert_type(m6.astype(jnp.bfloat16),
                                     jnp.uint16).astype(jnp.uint32)
        w = (b[:, 0::2] << 16) | b[:, 1::2]
        return lax.bitcast_convert_type(w, jnp.int32).reshape(-1)

    h1, t1lo, t1hi, ad6lo, ad6hi, asp, adp = _tc_prep1(
        x, W1, A1s, A1d, A1s_lo, A1s_hi, A1d_lo, A1d_hi)
    t1 = jnp.concatenate([t1lo, t1hi], axis=0)
    ad_all = jnp.concatenate([pack_pairs(ad6lo[:, :_HH]),
                              pack_pairs(ad6hi[:, :_HH])])
    u1f = _sc_edge_pass1(src, dst, t1, ad_all)
    u1 = (u1f[:_N], u1f[_N:])
    h2, sa2 = _tc_finish1(u1[0], u1[1], h1, asp, adp, R, b1r, W2, a2cols)
    sab = lax.bitcast_convert_type(sa2[:, :2].astype(jnp.bfloat16),
                                   jnp.uint16).astype(jnp.uint32)
    sa32 = lax.bitcast_convert_type((sab[:, 0] << 16) | sab[:, 1], jnp.int32)
    u2f, ud2f = _sc_edge_pass2(src, dst, sa32, h2)
    u2 = (u2f[:_N], u2f[_N:])
    ud2 = (ud2f[:_ND], ud2f[_ND:])
    ud0 = ud2[0].reshape(_ND * 8, 16)[:_N]
    ud1 = ud2[1].reshape(_ND * 8, 16)[:_N]
    out = _tc_finish2(u2[0], u2[1], ud0, ud1, h2, sa2, b2r)
    return (h1, t1lo, t1hi, ad6lo, ad6hi, asp, adp, u1, h2, sa2, sa32,
            u2, ud2, out)


# speedup vs baseline: 24.8973x; 1.1224x over previous
import dataclasses
import functools

import jax
import jax.numpy as jnp
from jax import lax
from jax.experimental import pallas as pl
from jax.experimental.pallas import tpu as pltpu
from jax.experimental.pallas import tpu_sc as plsc

_N = 10000
_E = 320000
_D = 128
_H1 = 12
_O1 = 16
_OUT = 128

_W1COL = _H1 * _O1
_HH = _H1 // 2
_HW = _HH * _O1

_NC = 2
_NS = 16
_B = 80
_RC = 80
_NCH = _N // _RC
_CPT = (_NCH + _NS - 1) // _NS
_ND = 1280

_RB = 1000


def _leaky_exp(e):
    return jnp.exp(jnp.maximum(e, 0.2 * e))


def _dot(a, b):
    return jnp.dot(a, b, precision=lax.Precision.HIGHEST,
                   preferred_element_type=jnp.float32)


def _sc_compiler_params():
    cp = pltpu.CompilerParams()
    if "needs_layout_passes" in pltpu.CompilerParams.__dataclass_fields__:
        cp = dataclasses.replace(cp, needs_layout_passes=False)
    return cp


def _tc_prep1(x, W1, A1s, A1d, A1s_lo, A1s_hi, A1d_lo, A1d_hi):
    def body(x_ref, w_ref, asf_ref, adf_ref, aslo_ref, ashi_ref, adlo_ref,
             adhi_ref, h_ref, t1lo_ref, t1hi_ref, ad6lo_ref, ad6hi_ref,
             asp_ref, adp_ref):
        h = _dot(x_ref[...], w_ref[...])
        h_ref[...] = h
        asp_ref[...] = _dot(h, asf_ref[...])
        adp_ref[...] = _dot(h, adf_ref[...])
        z16 = jnp.zeros((h.shape[0], 16), jnp.float32)
        aslo = _dot(h, aslo_ref[...])
        ashi = _dot(h, ashi_ref[...])
        t1lo_ref[...] = jnp.concatenate([h[:, :_HW], aslo, z16], axis=1)
        t1hi_ref[...] = jnp.concatenate([h[:, _HW:_W1COL], ashi, z16], axis=1)
        ad6lo_ref[...] = _dot(h, adlo_ref[...])
        ad6hi_ref[...] = _dot(h, adhi_ref[...])

    full = lambda shape: pl.BlockSpec(shape, lambda i: (0, 0))
    row = lambda cols: pl.BlockSpec((_RB, cols), lambda i: (i, 0))
    return pl.pallas_call(
        body,
        grid=(_N // _RB,),
        in_specs=[row(_D), full((_D, _W1COL)), full((_W1COL, 16)),
                  full((_W1COL, 16)), full((_W1COL, 16)), full((_W1COL, 16)),
                  full((_W1COL, 8)), full((_W1COL, 8))],
        out_specs=[row(_W1COL), row(128), row(128), row(8), row(8),
                   row(16), row(16)],
        out_shape=[
            jax.ShapeDtypeStruct((_N, _W1COL), jnp.float32),
            jax.ShapeDtypeStruct((_N, 128), jnp.float32),
            jax.ShapeDtypeStruct((_N, 128), jnp.float32),
            jax.ShapeDtypeStruct((_N, 8), jnp.float32),
            jax.ShapeDtypeStruct((_N, 8), jnp.float32),
            jax.ShapeDtypeStruct((_N, 16), jnp.float32),
            jax.ShapeDtypeStruct((_N, 16), jnp.float32),
        ],
    )(x, W1, A1s, A1d, A1s_lo, A1s_hi, A1d_lo, A1d_hi)


def _tc_finish1(ulo, uhi, h1, asp, adp, R, b1r, W2, a2cols):
    def body(ulo_ref, uhi_ref, h_ref, as_ref, ad_ref, r_ref, b_ref, w2_ref,
             a2_ref, h2_ref, sa_ref):
        e = as_ref[...] + ad_ref[...]
        exl = _leaky_exp(e)
        rm = r_ref[...]
        ulob = ulo_ref[...]
        uhib = uhi_ref[...]
        dent = jnp.concatenate(
            [ulob[:, _HW:_HW + _HH], uhib[:, _HW:_HW + _HH],
             jnp.zeros((ulob.shape[0], 4), jnp.float32)], axis=1) + exl
        numer = (jnp.concatenate([ulob[:, :_HW], uhib[:, :_HW]], axis=1)
                 + h_ref[...] * _dot(exl, rm))
        out1 = numer / (_dot(dent, rm)
                        + 1e-16) + b_ref[...]
        x2 = jnp.maximum(out1, 0.0)
        h2 = _dot(x2, w2_ref[...])
        h2_ref[...] = h2
        sa_ref[...] = _dot(h2, a2_ref[...])

    full = lambda shape: pl.BlockSpec(shape, lambda i: (0, 0))
    row = lambda cols: pl.BlockSpec((_RB, cols), lambda i: (i, 0))
    return pl.pallas_call(
        body,
        grid=(_N // _RB,),
        in_specs=[row(128), row(128), row(_W1COL), row(16), row(16),
                  full((16, _W1COL)),
                  pl.BlockSpec((1, _W1COL), lambda i: (0, 0)),
                  full((_W1COL, _OUT)), full((_OUT, 8))],
        out_specs=[row(_OUT), row(8)],
        out_shape=[
            jax.ShapeDtypeStruct((_N, _OUT), jnp.float32),
            jax.ShapeDtypeStruct((_N, 8), jnp.float32),
        ],
    )(ulo, uhi, h1, asp, adp, R, b1r, W2, a2cols)


def _tc_finish2(u20, u21, ud0, ud1, h2, sa2, b2r):
    def body(u20_ref, u21_ref, ud0_ref, ud1_ref, h2_ref, sa_ref, b_ref,
             out_ref):
        sa = sa_ref[...]
        e = sa[:, 0:1] + sa[:, 1:2]
        exl = _leaky_exp(e)
        denom = ud0_ref[:, 0:1] + ud1_ref[:, 0:1] + exl + 1e-16
        numer = u20_ref[...] + u21_ref[...] + h2_ref[...] * exl
        out_ref[...] = numer / denom + b_ref[...]

    row = lambda cols: pl.BlockSpec((_RB, cols), lambda i: (i, 0))
    return pl.pallas_call(
        body,
        grid=(_N // _RB,),
        in_specs=[row(_OUT), row(_OUT), row(16), row(16), row(_OUT),
                  row(8), pl.BlockSpec((1, _OUT), lambda i: (0, 0))],
        out_specs=row(_OUT),
        out_shape=jax.ShapeDtypeStruct((_N, _OUT), jnp.float32),
    )(u20, u21, ud0, ud1, h2, sa2, b2r)


def _sc_edge_pass1(src, dst, t1, ad_all):
    mesh = plsc.VectorSubcoreMesh(core_axis_name="c", subcore_axis_name="s")
    ept = _E // _NS
    nb = ept // _B

    @functools.partial(
        pl.kernel, mesh=mesh,
        compiler_params=_sc_compiler_params(),
        out_type=jax.ShapeDtypeStruct((_NC * _N, 128), jnp.float32),
        scratch_types=[
            pltpu.VMEM_SHARED((_N, 128), jnp.float32),
            pltpu.VMEM((3 * _N,), jnp.int32),
            pltpu.VMEM((_B,), jnp.int32),
            pltpu.VMEM((_B,), jnp.int32),
            pltpu.VMEM((_B,), jnp.int32),
            pltpu.VMEM((_B, 128), jnp.float32),
            pltpu.VMEM((_B * 16,), jnp.float32),
        ])
    def k(src_hbm, dst_hbm, t1_hbm, ad_hbm, u_hbm,
          acc, ad32, srcb, srcb2, dstb, msgb, exflat):
        sc = lax.axis_index("c")
        tid = lax.axis_index("s")

        pltpu.sync_copy(ad_hbm.at[pl.ds(sc * (3 * _N), 3 * _N)], ad32)

        @pl.loop(0, _RC)
        def _(r):
            for c in range(8):
                msgb[r, pl.ds(c * 16, 16)] = jnp.zeros((16,), jnp.float32)

        @pl.loop(0, _CPT)
        def _(kk):
            ch = kk * _NS + tid

            @pl.when(ch < _NCH)
            def _():
                pltpu.sync_copy(msgb, acc.at[pl.ds(ch * _RC, _RC)])

        plsc.subcore_barrier()

        lanes = lax.iota(jnp.int32, 16)
        lanes16 = lanes * 16
        hm = jnp.minimum(lanes, _HH - 1)
        himask = jnp.full((16,), -65536, jnp.int32)
        sixteen = jnp.full((16,), 16, jnp.int32)
        base = tid * ept

        rowoff = jnp.full((16,), sc * _N, jnp.int32)

        @pl.loop(0, nb)
        def _(blk):
            off = base + blk * _B
            pltpu.sync_copy(src_hbm.at[pl.ds(off, _B)], srcb)
            pltpu.sync_copy(dst_hbm.at[pl.ds(off, _B)], dstb)

            for c in range(_B // 16):
                srcb2[pl.ds(c * 16, 16)] = srcb[pl.ds(c * 16, 16)] + rowoff

            pltpu.sync_copy(t1_hbm.at[srcb2], msgb)

            for c in range(_B // 16):
                ev = lanes + c * 16
                dv3 = dstb[pl.ds(c * 16, 16)] * 3
                for kw in range(3):
                    gd = plsc.load_gather(ad32, [dv3 + kw])
                    adhi = lax.bitcast_convert_type(gd & himask, jnp.float32)
                    adlo = lax.bitcast_convert_type(
                        lax.shift_left(gd, sixteen), jnp.float32)
                    ashi = plsc.load_gather(
                        msgb, [ev, jnp.full((16,), _HW + 2 * kw, jnp.int32)])
                    aslo = plsc.load_gather(
                        msgb,
                        [ev, jnp.full((16,), _HW + 2 * kw + 1, jnp.int32)])
                    plsc.store_scatter(exflat, [lanes16 + (c * 256 + 2 * kw)],
                                       _leaky_exp(ashi + adhi))
                    plsc.store_scatter(exflat,
                                       [lanes16 + (c * 256 + 2 * kw + 1)],
                                       _leaky_exp(aslo + adlo))

            @pl.loop(0, _B // 2)
            def _(i):
                i0 = i * 2
                i1 = i0 + 1
                a16 = i0 * 16
                b16 = i1 * 16
                msgb[i0, pl.ds(_HW, 16)] = plsc.load_gather(exflat,
                                                            [hm + a16])
                msgb[i1, pl.ds(_HW, 16)] = plsc.load_gather(exflat,
                                                            [hm + b16])
                for hd in range(_HH):
                    bca = plsc.load_gather(
                        exflat, [jnp.full((16,), a16 + hd, jnp.int32)])
                    bcb = plsc.load_gather(
                        exflat, [jnp.full((16,), b16 + hd, jnp.int32)])
                    msgb[i0, pl.ds(hd * 16, 16)] = (
                        msgb[i0, pl.ds(hd * 16, 16)] * bca)
                    msgb[i1, pl.ds(hd * 16, 16)] = (
                        msgb[i1, pl.ds(hd * 16, 16)] * bcb)

            pltpu.sync_copy(msgb, acc.at[dstb], add=True)

        plsc.subcore_barrier()

        @pl.loop(0, _CPT)
        def _(kk):
            ch = kk * _NS + tid

            @pl.when(ch < _NCH)
            def _():
                r0 = ch * _RC
                pltpu.sync_copy(acc.at[pl.ds(r0, _RC)], msgb)
                pltpu.sync_copy(msgb, u_hbm.at[pl.ds(sc * _N + r0, _RC)])

    return k(src, dst, t1, ad_all)


def _sc_edge_pass2(src, dst, sa32, h2):
    mesh = plsc.VectorSubcoreMesh(core_axis_name="c", subcore_axis_name="s")
    ept = _E // (_NC * _NS)
    nb = ept // _B

    @functools.partial(
        pl.kernel, mesh=mesh,
        compiler_params=_sc_compiler_params(),
        out_type=[jax.ShapeDtypeStruct((_NC * _N, 128), jnp.float32),
                  jax.ShapeDtypeStruct((_NC * _ND, 128), jnp.float32)],
        scratch_types=[
            pltpu.VMEM_SHARED((_N, 128), jnp.float32),
            pltpu.VMEM_SHARED((_ND, 128), jnp.float32),
            pltpu.VMEM((_N,), jnp.int32),
            pltpu.VMEM((_B,), jnp.int32),
            pltpu.VMEM((_B,), jnp.int32),
            pltpu.VMEM((_B,), jnp.int32),
            pltpu.VMEM((_B,), jnp.float32),
            pltpu.VMEM((_B, 128), jnp.float32),
            pltpu.VMEM((_B, 128), jnp.float32),
        ])
    def k(src_hbm, dst_hbm, sa_hbm, h_hbm, u_hbm, ud_hbm,
          acc, accd, sa32v, srcb, dstb, dstq, exb, msgb, msgd):
        sc = lax.axis_index("c")
        tid = lax.axis_index("s")

        pltpu.sync_copy(sa_hbm, sa32v)

        @pl.loop(0, _RC)
        def _(r):
            for c in range(8):
                msgb[r, pl.ds(c * 16, 16)] = jnp.zeros((16,), jnp.float32)

        @pl.loop(0, _CPT)
        def _(kk):
            ch = kk * _NS + tid

            @pl.when(ch < _NCH)
            def _():
                pltpu.sync_copy(msgb, acc.at[pl.ds(ch * _RC, _RC)])

        pltpu.sync_copy(msgb, accd.at[pl.ds(tid * _RC, _RC)])

        plsc.subcore_barrier()

        himask = jnp.full((16,), -65536, jnp.int32)
        sixteen = jnp.full((16,), 16, jnp.int32)
        base = (sc * _NS + tid) * ept

        @pl.loop(0, nb)
        def _(blk):
            off = base + blk * _B
            pltpu.sync_copy(src_hbm.at[pl.ds(off, _B)], srcb)
            pltpu.sync_copy(dst_hbm.at[pl.ds(off, _B)], dstb)
            pltpu.sync_copy(h_hbm.at[srcb], msgb)

            for c in range(_B // 16):
                sv = srcb[pl.ds(c * 16, 16)]
                dv = dstb[pl.ds(c * 16, 16)]
                gs = plsc.load_gather(sa32v, [sv])
                gd = plsc.load_gather(sa32v, [dv])
                e = (lax.bitcast_convert_type(gs & himask, jnp.float32)
                     + lax.bitcast_convert_type(lax.shift_left(gd, sixteen),
                                                jnp.float32))
                exb[pl.ds(c * 16, 16)] = _leaky_exp(e)
                dstq[pl.ds(c * 16, 16)] = lax.shift_right_logical(dv, 3)

            @pl.loop(0, _B)
            def _(i):
                iv = jnp.full((16,), i, jnp.int32)
                bc = plsc.load_gather(exb, [iv])
                for c in range(_OUT // 16):
                    msgb[i, pl.ds(c * 16, 16)] = (
                        msgb[i, pl.ds(c * 16, 16)] * bc)
                dvb = plsc.load_gather(dstb, [iv])
                slot = (lax.reduce_max(dvb, axes=(0,)) & 7) * 16
                for c in range(8):
                    msgd[i, pl.ds(c * 16, 16)] = jnp.zeros((16,), jnp.float32)
                msgd[i, pl.ds(slot, 16)] = bc

            pltpu.sync_copy(msgb, acc.at[dstb], add=True)
            pltpu.sync_copy(msgd, accd.at[dstq], add=True)

        plsc.subcore_barrier()

        @pl.loop(0, _CPT)
        def _(kk):
            ch = kk * _NS + tid

            @pl.when(ch < _NCH)
            def _():
                r0 = ch * _RC
                pltpu.sync_copy(acc.at[pl.ds(r0, _RC)], msgb)
                pltpu.sync_copy(msgb, u_hbm.at[pl.ds(sc * _N + r0, _RC)])

        r0 = tid * _RC
        pltpu.sync_copy(accd.at[pl.ds(r0, _RC)], msgb)
        pltpu.sync_copy(msgb, ud_hbm.at[pl.ds(sc * _ND + r0, _RC)])

    return k(src, dst, sa32, h2)


def kernel(x, edge_index, W1, a_src1, a_dst1, b1, W2, a_src2, a_dst2, b2):
    return _run(x, edge_index, W1, a_src1, a_dst1, b1, W2, a_src2, a_dst2,
                b2)[-1]


def _run(x, edge_index, W1, a_src1, a_dst1, b1, W2, a_src2, a_dst2, b2):
    src = edge_index[0]
    dst = edge_index[1]

    rows = jnp.arange(_W1COL, dtype=jnp.int32)
    head_of_row = rows // _O1
    lane16 = jnp.arange(16, dtype=jnp.int32)

    def attn_mat(a, lo, hi, width):
        m = jnp.zeros((_W1COL, width), jnp.float32)
        sel = (head_of_row >= lo) & (head_of_row < hi)
        col = jnp.where(sel, head_of_row - lo, width - 1)
        val = jnp.where(sel, a.reshape(-1), 0.0)
        return m.at[rows, col].add(val)

    A1s = attn_mat(a_src1, 0, _H1, 16)
    A1d = attn_mat(a_dst1, 0, _H1, 16)
    A1s_lo = attn_mat(a_src1, 0, _HH, 16)
    A1s_hi = attn_mat(a_src1, _HH, _H1, 16)
    A1d_lo = attn_mat(a_dst1, 0, _HH, 8)
    A1d_hi = attn_mat(a_dst1, _HH, _H1, 8)
    R = (lane16[:, None] == head_of_row[None, :]).astype(jnp.float32)
    a2cols = jnp.zeros((_OUT, 8), jnp.float32)
    a2cols = a2cols.at[:, 0].set(a_src2[0]).at[:, 1].set(a_dst2[0])
    b1r = b1.reshape(1, _W1COL)
    b2r = b2.reshape(1, _OUT)

    def pack_pairs(m6):
        b = lax.bitcast_convert_type(m6.astype(jnp.bfloat16),
                                     jnp.uint16).astype(jnp.uint32)
        w = (b[:, 0::2] << 16) | b[:, 1::2]
        return lax.bitcast_convert_type(w, jnp.int32).reshape(-1)

    h1, t1lo, t1hi, ad6lo, ad6hi, asp, adp = _tc_prep1(
        x, W1, A1s, A1d, A1s_lo, A1s_hi, A1d_lo, A1d_hi)
    t1 = jnp.concatenate([t1lo, t1hi], axis=0)
    ad_all = jnp.concatenate([pack_pairs(ad6lo[:, :_HH]),
                              pack_pairs(ad6hi[:, :_HH])])
    u1f = _sc_edge_pass1(src, dst, t1, ad_all)
    u1 = (u1f[:_N], u1f[_N:])
    h2, sa2 = _tc_finish1(u1[0], u1[1], h1, asp, adp, R, b1r, W2, a2cols)
    sab = lax.bitcast_convert_type(sa2[:, :2].astype(jnp.bfloat16),
                                   jnp.uint16).astype(jnp.uint32)
    sa32 = lax.bitcast_convert_type((sab[:, 0] << 16) | sab[:, 1], jnp.int32)
    u2f, ud2f = _sc_edge_pass2(src, dst, sa32, h2)
    u2 = (u2f[:_N], u2f[_N:])
    ud2 = (ud2f[:_ND], ud2f[_ND:])
    ud0 = ud2[0].reshape(_ND * 8, 16)[:_N]
    ud1 = ud2[1].reshape(_ND * 8, 16)[:_N]
    out = _tc_finish2(u2[0], u2[1], ud0, ud1, h2, sa2, b2r)
    return (h1, t1lo, t1hi, ad6lo, ad6hi, asp, adp, u1, h2, sa2, sa32,
            u2, ud2, out)


# speedup vs baseline: 26.2977x; 1.0562x over previous
import dataclasses
import functools

import jax
import jax.numpy as jnp
from jax import lax
from jax.experimental import pallas as pl
from jax.experimental.pallas import tpu as pltpu
from jax.experimental.pallas import tpu_sc as plsc

_N = 10000
_E = 320000
_D = 128
_H1 = 12
_O1 = 16
_OUT = 128

_W1COL = _H1 * _O1
_HH = _H1 // 2
_HW = _HH * _O1

_NC = 2
_NS = 16
_B = 80
_RC = 80
_NCH = _N // _RC
_CPT = (_NCH + _NS - 1) // _NS
_ND = 1280

_RB = 1000


def _leaky_exp(e):
    return jnp.exp(jnp.maximum(e, 0.2 * e))


def _dot(a, b):
    return jnp.dot(a, b, precision=lax.Precision.HIGHEST,
                   preferred_element_type=jnp.float32)


def _sc_compiler_params():
    cp = pltpu.CompilerParams()
    if "needs_layout_passes" in pltpu.CompilerParams.__dataclass_fields__:
        cp = dataclasses.replace(cp, needs_layout_passes=False)
    return cp


def _tc_prep1(x, W1, A1s, A1d, A1s_lo, A1s_hi, A1d_lo, A1d_hi):
    def body(x_ref, w_ref, asf_ref, adf_ref, aslo_ref, ashi_ref, adlo_ref,
             adhi_ref, h_ref, t1lo_ref, t1hi_ref, ad6lo_ref, ad6hi_ref,
             asp_ref, adp_ref):
        h = _dot(x_ref[...], w_ref[...])
        h_ref[...] = h
        asp_ref[...] = _dot(h, asf_ref[...])
        adp_ref[...] = _dot(h, adf_ref[...])
        z16 = jnp.zeros((h.shape[0], 16), jnp.float32)
        aslo = _dot(h, aslo_ref[...])
        ashi = _dot(h, ashi_ref[...])
        t1lo_ref[...] = jnp.concatenate([h[:, :_HW], aslo, z16], axis=1)
        t1hi_ref[...] = jnp.concatenate([h[:, _HW:_W1COL], ashi, z16], axis=1)
        ad6lo_ref[...] = _dot(h, adlo_ref[...])
        ad6hi_ref[...] = _dot(h, adhi_ref[...])

    full = lambda shape: pl.BlockSpec(shape, lambda i: (0, 0))
    row = lambda cols: pl.BlockSpec((_RB, cols), lambda i: (i, 0))
    return pl.pallas_call(
        body,
        grid=(_N // _RB,),
        in_specs=[row(_D), full((_D, _W1COL)), full((_W1COL, 16)),
                  full((_W1COL, 16)), full((_W1COL, 16)), full((_W1COL, 16)),
                  full((_W1COL, 8)), full((_W1COL, 8))],
        out_specs=[row(_W1COL), row(128), row(128), row(8), row(8),
                   row(16), row(16)],
        out_shape=[
            jax.ShapeDtypeStruct((_N, _W1COL), jnp.float32),
            jax.ShapeDtypeStruct((_N, 128), jnp.float32),
            jax.ShapeDtypeStruct((_N, 128), jnp.float32),
            jax.ShapeDtypeStruct((_N, 8), jnp.float32),
            jax.ShapeDtypeStruct((_N, 8), jnp.float32),
            jax.ShapeDtypeStruct((_N, 16), jnp.float32),
            jax.ShapeDtypeStruct((_N, 16), jnp.float32),
        ],
    )(x, W1, A1s, A1d, A1s_lo, A1s_hi, A1d_lo, A1d_hi)


def _tc_finish1(ulo, uhi, h1, asp, adp, R, b1r, W2, a2cols):
    def body(ulo_ref, uhi_ref, h_ref, as_ref, ad_ref, r_ref, b_ref, w2_ref,
             a2_ref, h2_ref, sa_ref):
        e = as_ref[...] + ad_ref[...]
        exl = _leaky_exp(e)
        rm = r_ref[...]
        ulob = ulo_ref[...]
        uhib = uhi_ref[...]
        dent = jnp.concatenate(
            [ulob[:, _HW:_HW + _HH], uhib[:, _HW:_HW + _HH],
             jnp.zeros((ulob.shape[0], 4), jnp.float32)], axis=1) + exl
        numer = (jnp.concatenate([ulob[:, :_HW], uhib[:, :_HW]], axis=1)
                 + h_ref[...] * _dot(exl, rm))
        out1 = numer / (_dot(dent, rm)
                        + 1e-16) + b_ref[...]
        x2 = jnp.maximum(out1, 0.0)
        h2 = _dot(x2, w2_ref[...])
        h2_ref[...] = h2
        sa_ref[...] = _dot(h2, a2_ref[...])

    full = lambda shape: pl.BlockSpec(shape, lambda i: (0, 0))
    row = lambda cols: pl.BlockSpec((_RB, cols), lambda i: (i, 0))
    return pl.pallas_call(
        body,
        grid=(_N // _RB,),
        in_specs=[row(128), row(128), row(_W1COL), row(16), row(16),
                  full((16, _W1COL)),
                  pl.BlockSpec((1, _W1COL), lambda i: (0, 0)),
                  full((_W1COL, _OUT)), full((_OUT, 8))],
        out_specs=[row(_OUT), row(8)],
        out_shape=[
            jax.ShapeDtypeStruct((_N, _OUT), jnp.float32),
            jax.ShapeDtypeStruct((_N, 8), jnp.float32),
        ],
    )(ulo, uhi, h1, asp, adp, R, b1r, W2, a2cols)


def _tc_finish2(u20, u21, ud0, ud1, h2, sa2, b2r):
    def body(u20_ref, u21_ref, ud0_ref, ud1_ref, h2_ref, sa_ref, b_ref,
             out_ref):
        sa = sa_ref[...]
        e = sa[:, 0:1] + sa[:, 1:2]
        exl = _leaky_exp(e)
        denom = ud0_ref[:, 0:1] + ud1_ref[:, 0:1] + exl + 1e-16
        numer = u20_ref[...] + u21_ref[...] + h2_ref[...] * exl
        out_ref[...] = numer / denom + b_ref[...]

    row = lambda cols: pl.BlockSpec((_RB, cols), lambda i: (i, 0))
    return pl.pallas_call(
        body,
        grid=(_N // _RB,),
        in_specs=[row(_OUT), row(_OUT), row(16), row(16), row(_OUT),
                  row(8), pl.BlockSpec((1, _OUT), lambda i: (0, 0))],
        out_specs=row(_OUT),
        out_shape=jax.ShapeDtypeStruct((_N, _OUT), jnp.float32),
    )(u20, u21, ud0, ud1, h2, sa2, b2r)


def _sc_edge_pass1(src, dst, t1, ad_all):
    mesh = plsc.VectorSubcoreMesh(core_axis_name="c", subcore_axis_name="s")
    ept = _E // _NS
    nb = ept // _B

    @functools.partial(
        pl.kernel, mesh=mesh,
        compiler_params=_sc_compiler_params(),
        out_type=jax.ShapeDtypeStruct((_NC * _N, 128), jnp.float32),
        scratch_types=[
            pltpu.VMEM_SHARED((_N, 128), jnp.float32),
            pltpu.VMEM((3 * _N,), jnp.int32),
            pltpu.VMEM((_B,), jnp.int32),
            pltpu.VMEM((_B,), jnp.int32),
            pltpu.VMEM((_B,), jnp.int32),
            pltpu.VMEM((_B, 128), jnp.float32),
            pltpu.VMEM((_B * 16,), jnp.float32),
        ])
    def k(src_hbm, dst_hbm, t1_hbm, ad_hbm, u_hbm,
          acc, ad32, srcb, srcb2, dstb, msgb, exflat):
        sc = lax.axis_index("c")
        tid = lax.axis_index("s")

        pltpu.sync_copy(ad_hbm.at[pl.ds(sc * (3 * _N), 3 * _N)], ad32)

        @pl.loop(0, _RC)
        def _(r):
            for c in range(8):
                msgb[r, pl.ds(c * 16, 16)] = jnp.zeros((16,), jnp.float32)

        @pl.loop(0, _CPT)
        def _(kk):
            ch = kk * _NS + tid

            @pl.when(ch < _NCH)
            def _():
                pltpu.sync_copy(msgb, acc.at[pl.ds(ch * _RC, _RC)])

        plsc.subcore_barrier()

        lanes = lax.iota(jnp.int32, 16)
        lanes16 = lanes * 16
        hm = jnp.minimum(lanes, _HH - 1)
        himask = jnp.full((16,), -65536, jnp.int32)
        sixteen = jnp.full((16,), 16, jnp.int32)
        base = tid * ept

        rowoff = jnp.full((16,), sc * _N, jnp.int32)

        @pl.loop(0, nb)
        def _(blk):
            off = base + blk * _B
            pltpu.sync_copy(src_hbm.at[pl.ds(off, _B)], srcb)
            pltpu.sync_copy(dst_hbm.at[pl.ds(off, _B)], dstb)

            for c in range(_B // 16):
                srcb2[pl.ds(c * 16, 16)] = srcb[pl.ds(c * 16, 16)] + rowoff

            pltpu.sync_copy(t1_hbm.at[srcb2], msgb)

            for c in range(_B // 16):
                ev = lanes + c * 16
                dv3 = dstb[pl.ds(c * 16, 16)] * 3
                for kw in range(3):
                    gd = plsc.load_gather(ad32, [dv3 + kw])
                    adhi = lax.bitcast_convert_type(gd & himask, jnp.float32)
                    adlo = lax.bitcast_convert_type(
                        lax.shift_left(gd, sixteen), jnp.float32)
                    ashi = plsc.load_gather(
                        msgb, [ev, jnp.full((16,), _HW + 2 * kw, jnp.int32)])
                    aslo = plsc.load_gather(
                        msgb,
                        [ev, jnp.full((16,), _HW + 2 * kw + 1, jnp.int32)])
                    plsc.store_scatter(exflat, [lanes16 + (c * 256 + 2 * kw)],
                                       _leaky_exp(ashi + adhi))
                    plsc.store_scatter(exflat,
                                       [lanes16 + (c * 256 + 2 * kw + 1)],
                                       _leaky_exp(aslo + adlo))

            @pl.loop(0, _B // 2)
            def _(i):
                i0 = i * 2
                i1 = i0 + 1
                a16 = i0 * 16
                b16 = i1 * 16
                msgb[i0, pl.ds(_HW, 16)] = plsc.load_gather(exflat,
                                                            [hm + a16])
                msgb[i1, pl.ds(_HW, 16)] = plsc.load_gather(exflat,
                                                            [hm + b16])
                for hd in range(_HH):
                    bca = plsc.load_gather(
                        exflat, [jnp.full((16,), a16 + hd, jnp.int32)])
                    bcb = plsc.load_gather(
                        exflat, [jnp.full((16,), b16 + hd, jnp.int32)])
                    msgb[i0, pl.ds(hd * 16, 16)] = (
                        msgb[i0, pl.ds(hd * 16, 16)] * bca)
                    msgb[i1, pl.ds(hd * 16, 16)] = (
                        msgb[i1, pl.ds(hd * 16, 16)] * bcb)

            pltpu.sync_copy(msgb, acc.at[dstb], add=True)

        plsc.subcore_barrier()

        @pl.loop(0, _CPT)
        def _(kk):
            ch = kk * _NS + tid

            @pl.when(ch < _NCH)
            def _():
                r0 = ch * _RC
                pltpu.sync_copy(acc.at[pl.ds(r0, _RC)], msgb)
                pltpu.sync_copy(msgb, u_hbm.at[pl.ds(sc * _N + r0, _RC)])

    return k(src, dst, t1, ad_all)


def _sc_edge_pass2(src, dst, sa32, h2):
    mesh = plsc.VectorSubcoreMesh(core_axis_name="c", subcore_axis_name="s")
    ept = _E // (_NC * _NS)
    nb = ept // _B

    @functools.partial(
        pl.kernel, mesh=mesh,
        compiler_params=_sc_compiler_params(),
        out_type=[jax.ShapeDtypeStruct((_NC * _N, 128), jnp.float32),
                  jax.ShapeDtypeStruct((_NC * _ND, 128), jnp.float32)],
        scratch_types=[
            pltpu.VMEM_SHARED((_N, 128), jnp.float32),
            pltpu.VMEM_SHARED((_ND, 128), jnp.float32),
            pltpu.VMEM((_N,), jnp.int32),
            pltpu.VMEM((_B,), jnp.int32),
            pltpu.VMEM((_B,), jnp.int32),
            pltpu.VMEM((_B,), jnp.int32),
            pltpu.VMEM((_B,), jnp.float32),
            pltpu.VMEM((_B, 128), jnp.float32),
            pltpu.VMEM((_B, 128), jnp.float32),
        ])
    def k(src_hbm, dst_hbm, sa_hbm, h_hbm, u_hbm, ud_hbm,
          acc, accd, sa32v, srcb, dstb, dstq, exb, msgb, msgd):
        sc = lax.axis_index("c")
        tid = lax.axis_index("s")

        pltpu.sync_copy(sa_hbm, sa32v)

        @pl.loop(0, _RC)
        def _(r):
            for c in range(8):
                msgb[r, pl.ds(c * 16, 16)] = jnp.zeros((16,), jnp.float32)

        @pl.loop(0, _CPT)
        def _(kk):
            ch = kk * _NS + tid

            @pl.when(ch < _NCH)
            def _():
                pltpu.sync_copy(msgb, acc.at[pl.ds(ch * _RC, _RC)])

        pltpu.sync_copy(msgb, accd.at[pl.ds(tid * _RC, _RC)])

        plsc.subcore_barrier()

        himask = jnp.full((16,), -65536, jnp.int32)
        sixteen = jnp.full((16,), 16, jnp.int32)
        base = (sc * _NS + tid) * ept

        @pl.loop(0, nb)
        def _(blk):
            off = base + blk * _B
            pltpu.sync_copy(src_hbm.at[pl.ds(off, _B)], srcb)
            pltpu.sync_copy(dst_hbm.at[pl.ds(off, _B)], dstb)
            pltpu.sync_copy(h_hbm.at[srcb], msgb)

            for c in range(_B // 16):
                sv = srcb[pl.ds(c * 16, 16)]
                dv = dstb[pl.ds(c * 16, 16)]
                gs = plsc.load_gather(sa32v, [sv])
                gd = plsc.load_gather(sa32v, [dv])
                e = (lax.bitcast_convert_type(gs & himask, jnp.float32)
                     + lax.bitcast_convert_type(lax.shift_left(gd, sixteen),
                                                jnp.float32))
                exb[pl.ds(c * 16, 16)] = _leaky_exp(e)
                dstq[pl.ds(c * 16, 16)] = lax.shift_right_logical(dv, 3)

            @pl.loop(0, _B // 2)
            def _(i):
                i0 = i * 2
                i1 = i0 + 1
                iva = jnp.full((16,), i0, jnp.int32)
                ivb = jnp.full((16,), i1, jnp.int32)
                bca = plsc.load_gather(exb, [iva])
                bcb = plsc.load_gather(exb, [ivb])
                for c in range(_OUT // 16):
                    msgb[i0, pl.ds(c * 16, 16)] = (
                        msgb[i0, pl.ds(c * 16, 16)] * bca)
                    msgb[i1, pl.ds(c * 16, 16)] = (
                        msgb[i1, pl.ds(c * 16, 16)] * bcb)
                dva = plsc.load_gather(dstb, [iva])
                dvb = plsc.load_gather(dstb, [ivb])
                slota = (lax.reduce_max(dva, axes=(0,)) & 7) * 16
                slotb = (lax.reduce_max(dvb, axes=(0,)) & 7) * 16
                for c in range(8):
                    z = jnp.zeros((16,), jnp.float32)
                    msgd[i0, pl.ds(c * 16, 16)] = z
                    msgd[i1, pl.ds(c * 16, 16)] = z
                msgd[i0, pl.ds(slota, 16)] = bca
                msgd[i1, pl.ds(slotb, 16)] = bcb

            pltpu.sync_copy(msgb, acc.at[dstb], add=True)
            pltpu.sync_copy(msgd, accd.at[dstq], add=True)

        plsc.subcore_barrier()

        @pl.loop(0, _CPT)
        def _(kk):
            ch = kk * _NS + tid

            @pl.when(ch < _NCH)
            def _():
                r0 = ch * _RC
                pltpu.sync_copy(acc.at[pl.ds(r0, _RC)], msgb)
                pltpu.sync_copy(msgb, u_hbm.at[pl.ds(sc * _N + r0, _RC)])

        r0 = tid * _RC
        pltpu.sync_copy(accd.at[pl.ds(r0, _RC)], msgb)
        pltpu.sync_copy(msgb, ud_hbm.at[pl.ds(sc * _ND + r0, _RC)])

    return k(src, dst, sa32, h2)


def kernel(x, edge_index, W1, a_src1, a_dst1, b1, W2, a_src2, a_dst2, b2):
    return _run(x, edge_index, W1, a_src1, a_dst1, b1, W2, a_src2, a_dst2,
                b2)[-1]


def _run(x, edge_index, W1, a_src1, a_dst1, b1, W2, a_src2, a_dst2, b2):
    src = edge_index[0]
    dst = edge_index[1]

    rows = jnp.arange(_W1COL, dtype=jnp.int32)
    head_of_row = rows // _O1
    lane16 = jnp.arange(16, dtype=jnp.int32)

    def attn_mat(a, lo, hi, width):
        m = jnp.zeros((_W1COL, width), jnp.float32)
        sel = (head_of_row >= lo) & (head_of_row < hi)
        col = jnp.where(sel, head_of_row - lo, width - 1)
        val = jnp.where(sel, a.reshape(-1), 0.0)
        return m.at[rows, col].add(val)

    A1s = attn_mat(a_src1, 0, _H1, 16)
    A1d = attn_mat(a_dst1, 0, _H1, 16)
    A1s_lo = attn_mat(a_src1, 0, _HH, 16)
    A1s_hi = attn_mat(a_src1, _HH, _H1, 16)
    A1d_lo = attn_mat(a_dst1, 0, _HH, 8)
    A1d_hi = attn_mat(a_dst1, _HH, _H1, 8)
    R = (lane16[:, None] == head_of_row[None, :]).astype(jnp.float32)
    a2cols = jnp.zeros((_OUT, 8), jnp.float32)
    a2cols = a2cols.at[:, 0].set(a_src2[0]).at[:, 1].set(a_dst2[0])
    b1r = b1.reshape(1, _W1COL)
    b2r = b2.reshape(1, _OUT)

    def pack_pairs(m6):
        b = lax.bitcast_convert_type(m6.astype(jnp.bfloat16),
                                     jnp.uint16).astype(jnp.uint32)
        w = (b[:, 0::2] << 16) | b[:, 1::2]
        return lax.bitcast_convert_type(w, jnp.int32).reshape(-1)

    h1, t1lo, t1hi, ad6lo, ad6hi, asp, adp = _tc_prep1(
        x, W1, A1s, A1d, A1s_lo, A1s_hi, A1d_lo, A1d_hi)
    t1 = jnp.concatenate([t1lo, t1hi], axis=0)
    ad_all = jnp.concatenate([pack_pairs(ad6lo[:, :_HH]),
                              pack_pairs(ad6hi[:, :_HH])])
    u1f = _sc_edge_pass1(src, dst, t1, ad_all)
    u1 = (u1f[:_N], u1f[_N:])
    h2, sa2 = _tc_finish1(u1[0], u1[1], h1, asp, adp, R, b1r, W2, a2cols)
    sab = lax.bitcast_convert_type(sa2[:, :2].astype(jnp.bfloat16),
                                   jnp.uint16).astype(jnp.uint32)
    sa32 = lax.bitcast_convert_type((sab[:, 0] << 16) | sab[:, 1], jnp.int32)
    u2f, ud2f = _sc_edge_pass2(src, dst, sa32, h2)
    u2 = (u2f[:_N], u2f[_N:])
    ud2 = (ud2f[:_ND], ud2f[_ND:])
    ud0 = ud2[0].reshape(_ND * 8, 16)[:_N]
    ud1 = ud2[1].reshape(_ND * 8, 16)[:_N]
    out = _tc_finish2(u2[0], u2[1], ud0, ud1, h2, sa2, b2r)
    return (h1, t1lo, t1hi, ad6lo, ad6hi, asp, adp, u1, h2, sa2, sa32,
            u2, ud2, out)
